# Initial kernel scaffold; baseline (speedup 1.0000x reference)
import jax
import jax.numpy as jnp
from jax.experimental import pallas as pl


def kernel(node_features, edge_index, edge_attr, lattice, batch, W_ee1, b_ee1, W_ee2, b_ee2, W_m1, b_m1, W_m2, b_m2, W_u1, b_u1, W_u2, b_u2, W_la, b_la, W_al, b_al, ln_n_w, ln_n_b, ln_l_w, ln_l_b):
    raise NotImplementedError("write your pallas kernel here")


# speedup vs baseline: 3.6792x; 3.6792x over previous
import functools
import jax
import jax.numpy as jnp
from jax import lax
from jax.experimental import pallas as pl
from jax.experimental.pallas import tpu as pltpu
from jax.experimental.pallas import tpu_sc as plsc

_N, _E, _D, _ED, _B = 10000, 320000, 128, 20, 64
_K = 80
_NW = 32
_EPW = _E // _NW
_NCHUNK = _EPW // _K
_NSUB = 16
_ROWS = _N // _NSUB
_NB = 2000
_EB = 4000


def _dot(a, b):
    return lax.dot_general(a, b, (((1,), (0,)), ((), ())),
                           preferred_element_type=jnp.float32)


def _uv_body(nf_ref, wm1_ref, u_ref, v_ref):
    nf = nf_ref[...]
    u_ref[...] = _dot(nf, wm1_ref[0:_D, :])
    v_ref[...] = _dot(nf, wm1_ref[_D:2 * _D, :])


def _uv(nf, wm1):
    return pl.pallas_call(
        _uv_body,
        grid=(_N // _NB,),
        in_specs=[pl.BlockSpec((_NB, _D), lambda i: (i, 0)),
                  pl.BlockSpec((3 * _D, _D), lambda i: (0, 0))],
        out_specs=[pl.BlockSpec((_NB, _D), lambda i: (i, 0)),
                   pl.BlockSpec((_NB, _D), lambda i: (i, 0))],
        out_shape=[jax.ShapeDtypeStruct((_N, _D), jnp.float32),
                   jax.ShapeDtypeStruct((_N, _D), jnp.float32)],
    )(nf, wm1)


def _edge_body(ea_ref, wee1_ref, bee1_ref, wee2_ref, bee2_ref, wm1_ref,
               bm1_ref, c_ref):
    wm1c = wm1_ref[2 * _D:3 * _D, :]
    wc = _dot(wee2_ref[...], wm1c)
    bc = _dot(bee2_ref[...], wm1c) + bm1_ref[...]
    h1 = jax.nn.silu(_dot(ea_ref[...], wee1_ref[...]) + bee1_ref[...])
    c_ref[...] = _dot(h1, wc) + bc


def _edge(ea, wee1, bee1, wee2, bee2, wm1, bm1):
    full = lambda shape: pl.BlockSpec(shape, lambda i: tuple(0 for _ in shape))
    return pl.pallas_call(
        _edge_body,
        grid=(_E // _EB,),
        in_specs=[pl.BlockSpec((_EB, _ED), lambda i: (i, 0)),
                  full((_ED, _D)), full((1, _D)), full((_D, _D)),
                  full((1, _D)), full((3 * _D, _D)), full((1, _D))],
        out_specs=pl.BlockSpec((_EB, _D), lambda i: (i, 0)),
        out_shape=jax.ShapeDtypeStruct((_E, _D), jnp.float32),
    )(ea, wee1, bee1, wee2, bee2, wm1, bm1)


_sc_mesh = plsc.VectorSubcoreMesh(core_axis_name="c", subcore_axis_name="s")


@functools.partial(
    pl.kernel,
    mesh=_sc_mesh,
    compiler_params=pltpu.CompilerParams(use_tc_tiling_on_sc=False),
    out_type=jax.ShapeDtypeStruct((2, _N, _D), jnp.float32),
    scratch_types=[
        pltpu.VMEM((_K,), jnp.int32),
        pltpu.VMEM((_K,), jnp.int32),
        pltpu.VMEM((_K, _D), jnp.float32),
        pltpu.VMEM((_K, _D), jnp.float32),
        pltpu.VMEM((_K, _D), jnp.float32),
        pltpu.VMEM((_K, _D), jnp.float32),
        pltpu.VMEM_SHARED((_N, _D), jnp.float32),
        pltpu.SemaphoreType.DMA,
        pltpu.SemaphoreType.DMA,
        pltpu.SemaphoreType.DMA,
    ],
)
def _sc_agg(u_hbm, v_hbm, c_hbm, src_hbm, dst_hbm, z_hbm, out_hbm,
            idx_s, idx_d, gu, gv, cb, sbuf, p_sh, sem_u, sem_v, sem_c):
    cid = lax.axis_index("c")
    sid = lax.axis_index("s")
    wid = sid * 2 + cid

    pltpu.sync_copy(z_hbm.at[pl.ds(sid * _ROWS, _ROWS)],
                    p_sh.at[pl.ds(sid * _ROWS, _ROWS)])
    plsc.subcore_barrier()

    ebase = wid * _EPW

    def _chunk(t, carry):
        base = ebase + t * _K
        pltpu.sync_copy(src_hbm.at[pl.ds(base, _K)], idx_s)
        pltpu.sync_copy(dst_hbm.at[pl.ds(base, _K)], idx_d)
        cp_u = pltpu.async_copy(u_hbm.at[idx_d], gu, sem_u)
        cp_v = pltpu.async_copy(v_hbm.at[idx_s], gv, sem_v)
        cp_c = pltpu.async_copy(c_hbm.at[pl.ds(base, _K)], cb, sem_c)
        cp_u.wait()
        cp_v.wait()
        cp_c.wait()

        def _row(i, rcarry):
            for j in range(_D // 16):
                sl = pl.ds(16 * j, 16)
                x = gu[i, sl] + gv[i, sl] + cb[i, sl]
                sbuf[i, sl] = x / (1.0 + jnp.exp(-x))
            return rcarry

        lax.fori_loop(0, _K, _row, 0)
        pltpu.sync_copy(sbuf, p_sh.at[idx_d], add=True)
        return carry

    lax.fori_loop(0, _NCHUNK, _chunk, 0)
    plsc.subcore_barrier()
    pltpu.sync_copy(p_sh.at[pl.ds(sid * _ROWS, _ROWS)],
                    out_hbm.at[cid, pl.ds(sid * _ROWS, _ROWS)])


@functools.partial(
    pl.kernel,
    mesh=_sc_mesh,
    compiler_params=pltpu.CompilerParams(use_tc_tiling_on_sc=False),
    out_type=jax.ShapeDtypeStruct((2, _N, 16), jnp.float32),
    scratch_types=[
        pltpu.VMEM((_K,), jnp.int32),
        pltpu.VMEM((_K, 16), jnp.float32),
        pltpu.VMEM_SHARED((_N, 16), jnp.float32),
    ],
)
def _sc_deg(dst_hbm, z_hbm, ones_hbm, out_hbm, idx_d, ones_v, d_sh):
    cid = lax.axis_index("c")
    sid = lax.axis_index("s")
    wid = sid * 2 + cid

    pltpu.sync_copy(z_hbm.at[pl.ds(sid * _ROWS, _ROWS)],
                    d_sh.at[pl.ds(sid * _ROWS, _ROWS)])
    pltpu.sync_copy(ones_hbm, ones_v)
    plsc.subcore_barrier()

    ebase = wid * _EPW

    def _chunk(t, carry):
        base = ebase + t * _K
        pltpu.sync_copy(dst_hbm.at[pl.ds(base, _K)], idx_d)
        pltpu.sync_copy(ones_v, d_sh.at[idx_d], add=True)
        return carry

    lax.fori_loop(0, _NCHUNK, _chunk, 0)
    plsc.subcore_barrier()
    pltpu.sync_copy(d_sh.at[pl.ds(sid * _ROWS, _ROWS)],
                    out_hbm.at[cid, pl.ds(sid * _ROWS, _ROWS)])


def _node_body(pp_ref, dg_ref, nf_ref, b3_ref, lat_ref, wm2_ref, bm2_ref,
               wu1_ref, bu1_ref, wu2_ref, bu2_ref, wla_ref, bla_ref, wal_ref,
               bal_ref, lnnw_ref, lnnb_ref, lnlw_ref, lnlb_ref,
               nfo_ref, seg_ref, lato_ref):
    P = pp_ref[0, :, :] + pp_ref[1, :, :]
    deg = dg_ref[0, :, 0:1] + dg_ref[1, :, 0:1]
    aggr = _dot(P, wm2_ref[...]) + deg * bm2_ref[...]
    nf = nf_ref[...]
    h = jax.nn.silu(_dot(nf, wu1_ref[0:_D, :]) +
                    _dot(aggr, wu1_ref[_D:2 * _D, :]) + bu1_ref[...])
    upd = _dot(h, wu2_ref[...]) + bu2_ref[...]
    lat9 = lat_ref[...]
    lat_info = _dot(lat9, wla_ref[...]) + bla_ref[...]
    bvec = b3_ref[0, 0, :]
    oh = (bvec[:, None] ==
          lax.broadcasted_iota(jnp.int32, (_NB, _B), 1)).astype(jnp.float32)
    nf2 = nf + upd + _dot(oh, lat_info)
    m = jnp.mean(nf2, axis=-1, keepdims=True)
    var = jnp.mean((nf2 - m) * (nf2 - m), axis=-1, keepdims=True)
    nfo_ref[...] = (nf2 - m) / jnp.sqrt(var + 1e-5) * lnnw_ref[...] + lnnb_ref[...]

    ext = jnp.concatenate([nf2, jnp.ones_like(nf2)], axis=1)
    contrib = lax.dot_general(oh, ext, (((0,), (0,)), ((), ())),
                              preferred_element_type=jnp.float32)

    @pl.when(pl.program_id(0) == 0)
    def _():
        seg_ref[...] = jnp.zeros_like(seg_ref)

    seg_ref[...] += contrib

    @pl.when(pl.program_id(0) == pl.num_programs(0) - 1)
    def _():
        seg = seg_ref[...]
        cnt = jnp.clip(seg[:, _D:_D + 1], 1.0, None)
        atom = seg[:, 0:_D] / cnt
        lat2 = lat9 + _dot(atom, wal_ref[...]) + bal_ref[...]
        m2 = jnp.mean(lat2, axis=-1, keepdims=True)
        v2 = jnp.mean((lat2 - m2) * (lat2 - m2), axis=-1, keepdims=True)
        lato_ref[...] = (lat2 - m2) / jnp.sqrt(v2 + 1e-5) * lnlw_ref[...] + lnlb_ref[...]


def _node(pp, dg, nf, b3, lat9, wm2, bm2, wu1, bu1, wu2, bu2, wla, bla, wal,
          bal, lnnw, lnnb, lnlw, lnlb):
    full = lambda shape: pl.BlockSpec(shape, lambda i: tuple(0 for _ in shape))
    return pl.pallas_call(
        _node_body,
        grid=(_N // _NB,),
        in_specs=[pl.BlockSpec((2, _NB, _D), lambda i: (0, i, 0)),
                  pl.BlockSpec((2, _NB, 16), lambda i: (0, i, 0)),
                  pl.BlockSpec((_NB, _D), lambda i: (i, 0)),
                  pl.BlockSpec((1, 1, _NB), lambda i: (i, 0, 0)),
                  full((_B, 9)),
                  full((_D, _D)), full((1, _D)),
                  full((2 * _D, _D)), full((1, _D)),
                  full((_D, _D)), full((1, _D)),
                  full((9, _D)), full((1, _D)),
                  full((_D, 9)), full((1, 9)),
                  full((1, _D)), full((1, _D)), full((1, 9)), full((1, 9))],
        out_specs=[pl.BlockSpec((_NB, _D), lambda i: (i, 0)),
                   pl.BlockSpec((_B, 2 * _D), lambda i: (0, 0)),
                   pl.BlockSpec((_B, 9), lambda i: (0, 0))],
        out_shape=[jax.ShapeDtypeStruct((_N, _D), jnp.float32),
                   jax.ShapeDtypeStruct((_B, 2 * _D), jnp.float32),
                   jax.ShapeDtypeStruct((_B, 9), jnp.float32)],
    )(pp, dg, nf, b3, lat9, wm2, bm2, wu1, bu1, wu2, bu2, wla, bla, wal, bal,
      lnnw, lnnb, lnlw, lnlb)


def kernel(node_features, edge_index, edge_attr, lattice, batch,
           W_ee1, b_ee1, W_ee2, b_ee2, W_m1, b_m1, W_m2, b_m2,
           W_u1, b_u1, W_u2, b_u2, W_la, b_la, W_al, b_al,
           ln_n_w, ln_n_b, ln_l_w, ln_l_b):
    r = lambda x: x.reshape(1, -1)
    src = edge_index[0]
    dst = edge_index[1]
    lat9 = lattice.reshape(_B, 9)
    u, v = _uv(node_features, W_m1)
    c = _edge(edge_attr, W_ee1, r(b_ee1), W_ee2, r(b_ee2), W_m1, r(b_m1))
    zinit = jnp.zeros((_N, _D), jnp.float32)
    z16 = jnp.zeros((_N, 16), jnp.float32)
    ones16 = jnp.ones((_K, 16), jnp.float32)
    pp = _sc_agg(u, v, c, src, dst, zinit)
    dg = _sc_deg(dst, z16, ones16)
    b3 = batch.reshape(_N // _NB, 1, _NB)
    nf_out, _, lat_out = _node(pp, dg, node_features, b3, lat9, W_m2, r(b_m2),
                               W_u1, r(b_u1), W_u2, r(b_u2), W_la, r(b_la),
                               W_al, r(b_al), r(ln_n_w), r(ln_n_b),
                               r(ln_l_w), r(ln_l_b))
    return nf_out, lat_out.reshape(_B, 3, 3)
